# baseline (speedup 1.0000x reference)
import dataclasses

import jax
import jax.numpy as jnp
from jax import lax
from jax.experimental import pallas as pl
from jax.experimental.pallas import tpu as pltpu
from jax.experimental.pallas import tpu_sc as plsc

_S = 512
_C = 96
_N = 2 * _C
_HALF = 10
_INV_BW = 1.0 / 21.0
_G = 8
_GRAN = 32
_NROWS = _N * _S * _S // _GRAN


def _sc_band_kernel(xg_hbm, idx_hbm, fpb_hbm, band_hbm,
                    idx_v, rows_v, fpb_v, band_v, sem):
    wid = lax.axis_index("s") * 2 + lax.axis_index("c")
    iot = lax.iota(jnp.int32, 16)
    zeros16 = jnp.zeros((16,), jnp.float32)

    @pl.loop(0, 6)
    def _(t):
        p = wid * 6 + t
        pltpu.sync_copy(idx_hbm.at[p], idx_v)
        copies = [
            pltpu.async_copy(xg_hbm.at[idx_v.at[k]], rows_v.at[k], sem)
            for k in range(8)
        ]
        pltpu.sync_copy(fpb_hbm.at[p], fpb_v)
        for i in range(35):
            band_v[pl.ds(16 * i, 16)] = zeros16
        for cp in copies:
            cp.wait()

        @pl.loop(0, 32)
        def _(g):
            rbase = g * 16
            fpv = fpb_v[pl.ds(rbase, 16)]
            for j in range(21):
                colv = iot + (rbase - _HALF + j)
                m = (colv >= 0) & (colv < _S)
                fps = jnp.maximum(fpv + j, 0)
                k_idx = jnp.right_shift(fps, 12)
                r_idx = jnp.bitwise_and(jnp.right_shift(fps, 5), 127)
                c_idx = jnp.bitwise_and(fps, 31)
                v = plsc.load_gather(rows_v, [k_idx, r_idx, c_idx])
                plsc.addupdate_scatter(band_v, [colv + 16],
                                       jnp.where(m, v, 0.0))

        pltpu.sync_copy(band_v.at[pl.ds(16, _S)], band_hbm.at[p])


def _attn_kernel(band_ref, cw_ref, pw_ref, pb_ref, out_ref):
    band = band_ref[...]
    cw = cw_ref[...]
    bp = jnp.pad(band, ((0, 0), (3, 3)))
    attn = cw[:, 0:1] * bp[:, 0:_S]
    for k in range(1, 7):
        attn = attn + cw[:, k:k + 1] * bp[:, k:k + _S]
    pw = pw_ref[...]
    a0 = jnp.dot(pw, attn[:_C], preferred_element_type=jnp.float32)
    a1 = jnp.dot(pw, attn[_C:], preferred_element_type=jnp.float32)
    attn = jnp.concatenate([a0, a1], axis=0) + pb_ref[...]
    m = jnp.max(attn, axis=1, keepdims=True)
    e = jnp.exp(attn - m)
    out_ref[...] = e / jnp.sum(e, axis=1, keepdims=True)


def _copy_sub_kernel(x_ref, attn_ref, y_ref):
    xb = x_ref[...]
    at = attn_ref[...]
    r = jax.lax.broadcasted_iota(jnp.int32, (1, _S, _S), 1)
    c = jax.lax.broadcasted_iota(jnp.int32, (1, _S, _S), 2)
    y_ref[...] = jnp.where(r == c, at * xb, xb)


def _band_indices():
    p = jnp.arange(_N, dtype=jnp.int32)[:, None]
    r = jnp.arange(_S, dtype=jnp.int32)[None, :]
    qs = p * (_S * _S) + 513 * r - _HALF
    g0 = jnp.maximum(qs, 0) // _GRAN
    g1 = jnp.minimum(g0 + 1, _NROWS - 1)
    idx = jnp.stack([g0, g1], axis=-1).reshape(_N, 8, 128)
    fpb = 64 * r + (qs - _GRAN * g0)
    return idx.astype(jnp.int32), fpb.astype(jnp.int32)


def kernel(x, conv_w, point_w, point_b):
    b, c, h, w = x.shape
    x3 = x.reshape(_N, _S, _S)
    xg = x.reshape(_NROWS, _GRAN)
    idx, fpb = _band_indices()

    mesh = plsc.VectorSubcoreMesh(core_axis_name="c", subcore_axis_name="s")
    cp = pltpu.CompilerParams()
    if "needs_layout_passes" in pltpu.CompilerParams.__dataclass_fields__:
        cp = dataclasses.replace(cp, needs_layout_passes=False,
                                 use_tc_tiling_on_sc=False)
    sc_band = pl.kernel(
        _sc_band_kernel,
        out_type=jax.ShapeDtypeStruct((_N, _S), jnp.float32),
        mesh=mesh,
        scratch_types=[
            pltpu.VMEM((8, 128), jnp.int32),
            pltpu.VMEM((8, 128, _GRAN), jnp.float32),
            pltpu.VMEM((_S,), jnp.int32),
            pltpu.VMEM((560,), jnp.float32),
            pltpu.SemaphoreType.DMA,
        ],
        compiler_params=cp,
    )
    band = sc_band(xg, idx, fpb)

    cw = jnp.tile(conv_w.reshape(_C, 7), (2, 1)) * _INV_BW
    pw = point_w.reshape(_C, _C)
    pb = jnp.tile(point_b.reshape(_C, 1), (2, 1))

    attn = pl.pallas_call(
        _attn_kernel,
        out_shape=jax.ShapeDtypeStruct((_N, _S), jnp.float32),
    )(band, cw, pw, pb)

    at3 = attn.reshape(_N, 1, _S)
    out = pl.pallas_call(
        _copy_sub_kernel,
        grid=(_N // _G,),
        in_specs=[
            pl.BlockSpec((_G, _S, _S), lambda n: (n, 0, 0)),
            pl.BlockSpec((_G, 1, _S), lambda n: (n, 0, 0)),
        ],
        out_specs=pl.BlockSpec((_G, _S, _S), lambda n: (n, 0, 0)),
        out_shape=jax.ShapeDtypeStruct((_N, _S, _S), jnp.float32),
    )(x3, at3)

    return out.reshape(b, c, h, w)
